# baseline (speedup 1.0000x reference)
import math

import numpy as np

import jax
import jax.numpy as jnp
from jax import lax
from jax.experimental import pallas as pl
from jax.experimental.pallas import tpu as pltpu, tpu_sc as plsc

N_TOKENS = 100000
D = 128
B = 1024
S = 200

NC = 2
NS = 16
NW = NC * NS
L = 16

CHUNK = 64
NCH = B // CHUNK
TCH = S * NCH
WCH = TCH // NW
NB = 8
GL = 6
SCALE = math.sqrt(float(D))

S_MAX = 7
ROWS_W = S_MAX * NCH

_BASE_S = [(WCH * w) // NCH for w in range(NW)]
_ROW_MAP = np.minimum(
    np.array(_BASE_S)[:, None] * NCH + np.arange(ROWS_W)[None, :], TCH - 1)
_PE_MAP = np.minimum(np.array(_BASE_S)[:, None] + np.arange(S_MAX)[None, :],
                     S - 1)


def _sc_body(table_hbm, idx_hbm, pe_hbm, out_hbm,
             idx_all, pe_all, *rest):
    wid = lax.axis_index("s") * NC + lax.axis_index("c")
    base_s = (WCH * wid) // NCH
    r = (WCH * wid) % NCH

    bufs = rest[:NB]
    gsem = rest[NB:2 * NB]
    ssem = rest[2 * NB:]

    pltpu.sync_copy(idx_hbm.at[wid], idx_all)
    pltpu.sync_copy(pe_hbm.at[wid], pe_all)

    g = [None] * NB
    st = [None] * NB
    for t in range(GL):
        g[t] = pltpu.async_copy(table_hbm.at[idx_all.at[r + t]],
                                bufs[t], gsem[t])
    for t in range(WCH):
        b = t % NB
        row = r + t
        li = row // NCH
        s = base_s + li
        c = row % NCH
        g[b].wait()
        pe_vs = [pe_all[li, pl.ds(L * j, L)] for j in range(D // L)]

        def row_body(q, carry3, _buf=bufs[b], _pe=pe_vs):
            for u in range(2):
                for j in range(D // L):
                    v = _buf[2 * q + u, pl.ds(L * j, L)]
                    _buf[2 * q + u, pl.ds(L * j, L)] = v * SCALE + _pe[j]
            return carry3

        lax.fori_loop(0, CHUNK // 2, row_body, 0)
        st[b] = pltpu.async_copy(
            bufs[b],
            out_hbm.at[s, pl.ds(pl.multiple_of(c * CHUNK, CHUNK), CHUNK)],
            ssem[b])
        if t + GL < WCH:
            b3 = (t + GL) % NB
            if st[b3] is not None:
                st[b3].wait()
            g[b3] = pltpu.async_copy(table_hbm.at[idx_all.at[r + t + GL]],
                                     bufs[b3], gsem[b3])
    for b in range(NB):
        st[b].wait()


def kernel(x, table, pe):
    idx_flat = jnp.transpose(x).astype(jnp.int32).reshape(TCH, CHUNK)
    idx_w = idx_flat[_ROW_MAP]
    pe_w = pe[_PE_MAP, 0, :]

    mesh = plsc.VectorSubcoreMesh(
        core_axis_name="c", subcore_axis_name="s",
        num_cores=NC, num_subcores=NS,
    )
    out = pl.kernel(
        _sc_body,
        out_type=jax.ShapeDtypeStruct((S, B, D), jnp.float32),
        mesh=mesh,
        scratch_types=(
            [pltpu.VMEM((ROWS_W, CHUNK), jnp.int32),
             pltpu.VMEM((S_MAX, D), jnp.float32)]
            + [pltpu.VMEM((CHUNK, D), jnp.float32)] * NB
            + [pltpu.SemaphoreType.DMA] * (2 * NB)
        ),
    )(table, idx_w, pe_w)
    return out


# speedup vs baseline: 1.0664x; 1.0664x over previous
import math

import numpy as np

import jax
import jax.numpy as jnp
from jax import lax
from jax.experimental import pallas as pl
from jax.experimental.pallas import tpu as pltpu, tpu_sc as plsc

N_TOKENS = 100000
D = 128
B = 1024
S = 200

NC = 2
NS = 16
NW = NC * NS
L = 16

CHUNK = 128
NCH = B // CHUNK
TCH = S * NCH
WCH = TCH // NW
NB = 7
GL = 4
SCALE = math.sqrt(float(D))

S_MAX = 7
ROWS_W = S_MAX * NCH

_BASE_S = [(WCH * w) // NCH for w in range(NW)]
_ROW_MAP = np.minimum(
    np.array(_BASE_S)[:, None] * NCH + np.arange(ROWS_W)[None, :], TCH - 1)
_PE_MAP = np.minimum(np.array(_BASE_S)[:, None] + np.arange(S_MAX)[None, :],
                     S - 1)


def _sc_body(table_hbm, idx_hbm, pe_hbm, out_hbm,
             idx_all, pe_all, *rest):
    wid = lax.axis_index("s") * NC + lax.axis_index("c")
    base_s = (WCH * wid) // NCH
    r = (WCH * wid) % NCH

    bufs = rest[:NB]
    gsem = rest[NB:2 * NB]
    ssem = rest[2 * NB:]

    pltpu.sync_copy(idx_hbm.at[wid], idx_all)
    pltpu.sync_copy(pe_hbm.at[wid], pe_all)

    g = [None] * NB
    st = [None] * NB
    for t in range(GL):
        g[t] = pltpu.async_copy(table_hbm.at[idx_all.at[r + t]],
                                bufs[t], gsem[t])
    for t in range(WCH):
        b = t % NB
        row = r + t
        li = row // NCH
        s = base_s + li
        c = row % NCH
        g[b].wait()
        pe_vs = [pe_all[li, pl.ds(L * j, L)] for j in range(D // L)]

        def row_body(q, carry3, _buf=bufs[b], _pe=pe_vs):
            for u in range(2):
                for j in range(D // L):
                    v = _buf[2 * q + u, pl.ds(L * j, L)]
                    _buf[2 * q + u, pl.ds(L * j, L)] = v * SCALE + _pe[j]
            return carry3

        lax.fori_loop(0, CHUNK // 2, row_body, 0)
        st[b] = pltpu.async_copy(
            bufs[b],
            out_hbm.at[s, pl.ds(pl.multiple_of(c * CHUNK, CHUNK), CHUNK)],
            ssem[b])
        if t + GL < WCH:
            b3 = (t + GL) % NB
            if st[b3] is not None:
                st[b3].wait()
            g[b3] = pltpu.async_copy(table_hbm.at[idx_all.at[r + t + GL]],
                                     bufs[b3], gsem[b3])
    for b in range(NB):
        st[b].wait()


def kernel(x, table, pe):
    idx_flat = jnp.transpose(x).astype(jnp.int32).reshape(TCH, CHUNK)
    idx_w = idx_flat[_ROW_MAP]
    pe_w = pe[_PE_MAP, 0, :]

    mesh = plsc.VectorSubcoreMesh(
        core_axis_name="c", subcore_axis_name="s",
        num_cores=NC, num_subcores=NS,
    )
    out = pl.kernel(
        _sc_body,
        out_type=jax.ShapeDtypeStruct((S, B, D), jnp.float32),
        mesh=mesh,
        scratch_types=(
            [pltpu.VMEM((ROWS_W, CHUNK), jnp.int32),
             pltpu.VMEM((S_MAX, D), jnp.float32)]
            + [pltpu.VMEM((CHUNK, D), jnp.float32)] * NB
            + [pltpu.SemaphoreType.DMA] * (2 * NB)
        ),
    )(table, idx_w, pe_w)
    return out


# speedup vs baseline: 1.1562x; 1.0842x over previous
import math

import numpy as np

import jax
import jax.numpy as jnp
from jax import lax
from jax.experimental import pallas as pl
from jax.experimental.pallas import tpu as pltpu, tpu_sc as plsc

N_TOKENS = 100000
D = 128
B = 1024
S = 200

NC = 2
NS = 16
NW = NC * NS
L = 16

CHUNK = 128
NCH = B // CHUNK
TCH = S * NCH
WCH = TCH // NW
NB = 7
GL = 4
SCALE = math.sqrt(float(D))

S_MAX = 7
ROWS_W = S_MAX * NCH

_BASE_S = [(WCH * w) // NCH for w in range(NW)]
_ROW_MAP = np.minimum(
    np.array(_BASE_S)[:, None] * NCH + np.arange(ROWS_W)[None, :], TCH - 1)
_PE_MAP = np.minimum(np.array(_BASE_S)[:, None] + np.arange(S_MAX)[None, :],
                     S - 1)


def _sc_body(table_hbm, idx_hbm, pe_hbm, out_hbm,
             idx_all, pe_all, *rest):
    wid = lax.axis_index("s") * NC + lax.axis_index("c")
    base_s = (WCH * wid) // NCH
    r = (WCH * wid) % NCH

    bufs = rest[:NB]
    gsem = rest[NB:2 * NB]
    ssem = rest[2 * NB:]

    row0 = pl.multiple_of(base_s * NCH, NCH)
    pltpu.sync_copy(idx_hbm.at[pl.ds(row0, ROWS_W)], idx_all)
    pltpu.sync_copy(pe_hbm.at[wid], pe_all)

    g = [None] * NB
    st = [None] * NB
    for t in range(GL):
        g[t] = pltpu.async_copy(table_hbm.at[idx_all.at[r + t]],
                                bufs[t], gsem[t])
    for t in range(WCH):
        b = t % NB
        row = r + t
        li = row // NCH
        s = base_s + li
        c = row % NCH
        g[b].wait()
        pe_vs = [pe_all[li, pl.ds(L * j, L)] for j in range(D // L)]

        def row_body(q, carry3, _buf=bufs[b], _pe=pe_vs):
            for u in range(2):
                for j in range(D // L):
                    v = _buf[2 * q + u, pl.ds(L * j, L)]
                    _buf[2 * q + u, pl.ds(L * j, L)] = v * SCALE + _pe[j]
            return carry3

        lax.fori_loop(0, CHUNK // 2, row_body, 0)
        st[b] = pltpu.async_copy(
            bufs[b],
            out_hbm.at[s, pl.ds(pl.multiple_of(c * CHUNK, CHUNK), CHUNK)],
            ssem[b])
        if t + GL < WCH:
            b3 = (t + GL) % NB
            if st[b3] is not None:
                st[b3].wait()
            g[b3] = pltpu.async_copy(table_hbm.at[idx_all.at[r + t + GL]],
                                     bufs[b3], gsem[b3])
    for b in range(NB):
        st[b].wait()


def kernel(x, table, pe):
    idx_flat = jnp.transpose(x).astype(jnp.int32).reshape(TCH, CHUNK)
    pe_w = pe[_PE_MAP, 0, :]

    mesh = plsc.VectorSubcoreMesh(
        core_axis_name="c", subcore_axis_name="s",
        num_cores=NC, num_subcores=NS,
    )
    out = pl.kernel(
        _sc_body,
        out_type=jax.ShapeDtypeStruct((S, B, D), jnp.float32),
        mesh=mesh,
        scratch_types=(
            [pltpu.VMEM((ROWS_W, CHUNK), jnp.int32),
             pltpu.VMEM((S_MAX, D), jnp.float32)]
            + [pltpu.VMEM((CHUNK, D), jnp.float32)] * NB
            + [pltpu.SemaphoreType.DMA] * (2 * NB)
        ),
    )(table, idx_flat, pe_w)
    return out
